# Initial kernel scaffold; baseline (speedup 1.0000x reference)
import jax
import jax.numpy as jnp
from jax.experimental import pallas as pl


def kernel(x, edge_index, batch, ratio, train_phase, W0, b0, W1, b1, W2, b2, lin1_W, lin1_b, lin2_W, lin2_b):
    raise NotImplementedError("write your pallas kernel here")


# speedup vs baseline: 6.6218x; 6.6218x over previous
import functools

import jax
import jax.numpy as jnp
from jax import lax
from jax.experimental import pallas as pl
from jax.experimental.pallas import tpu as pltpu
from jax.experimental.pallas import tpu_sc as plsc

NC = 2
NS = 16
NW = NC * NS
G = 64
CH = 80
INT_MIN = -2147483648

_mesh = plsc.VectorSubcoreMesh(core_axis_name="c", subcore_axis_name="s",
                               num_cores=NC, num_subcores=NS)
_cparams = pltpu.CompilerParams(needs_layout_passes=False)

_i16 = lambda: lax.iota(jnp.int32, 16)


def _splat_i(v):
    return jnp.full((16,), 1, jnp.int32) * v


def _splat_f(v):
    return jnp.full((16,), 1.0, jnp.float32) * v


def _sc_deg(dst, ew, n_pad):
    e = dst.shape[0]
    ew_per = e // NW
    nch = ew_per // CH
    nps = n_pad // NS

    @functools.partial(
        pl.kernel, mesh=_mesh, compiler_params=_cparams,
        out_type=jax.ShapeDtypeStruct((NC * n_pad,), jnp.float32),
        scratch_types=[
            pltpu.VMEM((1, CH), jnp.int32),
            pltpu.VMEM((CH,), jnp.float32),
            pltpu.VMEM((nps,), jnp.float32),
            pltpu.VMEM_SHARED((n_pad,), jnp.float32),
        ],
    )
    def k(dst_hbm, ew_hbm, out_hbm, idx_v, val_v, z_v, deg_sp):
        cid = lax.axis_index("c")
        sid = lax.axis_index("s")
        wid = cid * NS + sid
        for j in range(nps // 16):
            z_v[pl.ds(16 * j, 16)] = jnp.zeros((16,), jnp.float32)
        pltpu.sync_copy(z_v, deg_sp.at[pl.ds(sid * nps, nps)])
        plsc.subcore_barrier()

        def body(c, _):
            off = wid * ew_per + c * CH
            pltpu.sync_copy(dst_hbm.at[pl.ds(off, CH)], idx_v.at[0])
            pltpu.sync_copy(ew_hbm.at[pl.ds(off, CH)], val_v)
            pltpu.sync_copy(val_v, deg_sp.at[idx_v.at[0]], add=True)
            return 0

        lax.fori_loop(0, nch, body, 0)
        plsc.subcore_barrier()
        pltpu.sync_copy(deg_sp.at[pl.ds(sid * nps, nps)],
                        out_hbm.at[pl.ds(cid * n_pad + sid * nps, nps)])

    return k(dst, ew)


def _sc_rows(g_nodes, src, dst, ew, n_pad, scaled):
    e = src.shape[0]
    d = g_nodes.shape[1]
    ew_per = e // NW
    nch = ew_per // CH
    nps = n_pad // NS
    nf = d // 16

    scratch = [
        pltpu.VMEM((CH,), jnp.int32),
        pltpu.VMEM((1, CH), jnp.int32),
        pltpu.VMEM((CH, d), jnp.float32),
        pltpu.VMEM((CH,), jnp.float32),
        pltpu.VMEM_SHARED((n_pad, d), jnp.float32),
        pltpu.SemaphoreType.DMA,
    ]

    @functools.partial(
        pl.kernel, mesh=_mesh, compiler_params=_cparams,
        out_type=jax.ShapeDtypeStruct((NC * n_pad, d), jnp.float32),
        scratch_types=scratch,
    )
    def k(g_hbm, src_hbm, dst_hbm, ew_hbm, out_hbm, gidx_v, didx_v, rows_v,
          ew_v, acc_sp, sem):
        cid = lax.axis_index("c")
        sid = lax.axis_index("s")
        wid = cid * NS + sid

        def zrow(i, _):
            si = _splat_i(i)
            for f in range(nf):
                plsc.store_scatter(rows_v, [si, _i16() + 16 * f],
                                   jnp.zeros((16,), jnp.float32))
            return 0

        lax.fori_loop(0, CH, zrow, 0)
        for j in range(nps // CH):
            pltpu.sync_copy(rows_v, acc_sp.at[pl.ds(sid * nps + j * CH, CH)])
        plsc.subcore_barrier()

        def body(c, _):
            off = wid * ew_per + c * CH
            pltpu.sync_copy(src_hbm.at[pl.ds(off, CH)], gidx_v)
            pltpu.async_copy(g_hbm.at[gidx_v], rows_v, sem).wait()
            if scaled:
                pltpu.sync_copy(ew_hbm.at[pl.ds(off, CH)], ew_v)

                def scale(i, _):
                    si = _splat_i(i)
                    b = plsc.load_gather(ew_v, [si])
                    for f in range(nf):
                        col = _i16() + 16 * f
                        v = plsc.load_gather(rows_v, [si, col])
                        plsc.store_scatter(rows_v, [si, col], v * b)
                    return 0

                lax.fori_loop(0, CH, scale, 0)
            pltpu.sync_copy(dst_hbm.at[pl.ds(off, CH)], didx_v.at[0])
            pltpu.sync_copy(rows_v, acc_sp.at[didx_v.at[0]], add=True)
            return 0

        lax.fori_loop(0, nch, body, 0)
        plsc.subcore_barrier()
        pltpu.sync_copy(acc_sp.at[pl.ds(sid * nps, nps)],
                        out_hbm.at[pl.ds(cid * n_pad + sid * nps, nps)])

    return k(g_nodes, src, dst, ew)


def _sc_score(h, src, dst, batch):
    e = src.shape[0]
    n, d = h.shape
    ew_per = e // NW
    nch = ew_per // CH
    nf = d // 16

    @functools.partial(
        pl.kernel, mesh=_mesh, compiler_params=_cparams,
        out_type=(jax.ShapeDtypeStruct((e,), jnp.float32),
                  jax.ShapeDtypeStruct((e,), jnp.int32),
                  jax.ShapeDtypeStruct((NC * 2 * G,), jnp.int32)),
        scratch_types=[
            pltpu.VMEM((CH,), jnp.int32),
            pltpu.VMEM((CH,), jnp.int32),
            pltpu.VMEM((CH, d), jnp.float32),
            pltpu.VMEM((CH, d), jnp.float32),
            pltpu.VMEM((CH,), jnp.float32),
            pltpu.VMEM((CH,), jnp.int32),
            pltpu.VMEM((n,), jnp.int32),
            pltpu.VMEM((2 * G,), jnp.int32),
            pltpu.VMEM((1, 2 * G), jnp.int32),
            pltpu.VMEM_SHARED((2 * G,), jnp.int32),
            pltpu.SemaphoreType.DMA,
        ],
    )
    def k(h_hbm, src_hbm, dst_hbm, batch_hbm, sc_hbm, seg_hbm, cnt_hbm,
          sidx_v, didx_v, ra_v, rb_v, sco_v, sgo_v, bt_v, hist_v, idn_v,
          cnt_sp, sem):
        cid = lax.axis_index("c")
        sid = lax.axis_index("s")
        wid = cid * NS + sid
        pltpu.sync_copy(batch_hbm, bt_v)
        for j in range(2 * G // 16):
            hist_v[pl.ds(16 * j, 16)] = jnp.zeros((16,), jnp.int32)
            idn_v[0, pl.ds(16 * j, 16)] = _i16() + 16 * j

        @pl.when(sid == 0)
        def _():
            pltpu.sync_copy(hist_v, cnt_sp)

        plsc.subcore_barrier()

        ones16 = jnp.full((16,), 1, jnp.int32)
        lane0 = _i16() == 0

        def body(c, _):
            off = wid * ew_per + c * CH
            pltpu.sync_copy(src_hbm.at[pl.ds(off, CH)], sidx_v)
            pltpu.sync_copy(dst_hbm.at[pl.ds(off, CH)], didx_v)
            pltpu.async_copy(h_hbm.at[sidx_v], ra_v, sem).wait()
            pltpu.async_copy(h_hbm.at[didx_v], rb_v, sem).wait()
            for grp in range(CH // 16):
                sv = sidx_v[pl.ds(16 * grp, 16)]
                dv = didx_v[pl.ds(16 * grp, 16)]
                seg16 = plsc.load_gather(bt_v, [sv])
                sgo_v[pl.ds(16 * grp, 16)] = seg16
                plsc.addupdate_scatter(hist_v, [seg16], ones16)
                plsc.addupdate_scatter(hist_v, [seg16 + G], ones16,
                                       mask=sv < dv)

            def dot1(i, _):
                si = _splat_i(i)
                acc = jnp.zeros((16,), jnp.float32)
                for f in range(nf):
                    col = _i16() + 16 * f
                    acc = acc + (plsc.load_gather(ra_v, [si, col]) *
                                 plsc.load_gather(rb_v, [si, col]))
                s = jnp.sum(acc)
                plsc.store_scatter(sco_v, [si], _splat_f(s), mask=lane0)
                return 0

            lax.fori_loop(0, CH, dot1, 0)
            pltpu.sync_copy(sco_v, sc_hbm.at[pl.ds(off, CH)])
            pltpu.sync_copy(sgo_v, seg_hbm.at[pl.ds(off, CH)])
            return 0

        lax.fori_loop(0, nch, body, 0)
        pltpu.sync_copy(hist_v, cnt_sp.at[idn_v.at[0]], add=True)
        plsc.subcore_barrier()

        @pl.when(sid == 0)
        def _():
            pltpu.sync_copy(cnt_sp, cnt_hbm.at[pl.ds(cid * 2 * G, 2 * G)])

    return k(h, src, dst, batch)


def _sc_topk(key, seg, kper, scores):
    e = key.shape[0]
    ep = e // NS
    ngr = ep // 16

    @functools.partial(
        pl.kernel, mesh=_mesh, compiler_params=_cparams,
        out_type=(jax.ShapeDtypeStruct((e,), jnp.float32),
                  jax.ShapeDtypeStruct((e,), jnp.int32)),
        scratch_types=[
            pltpu.VMEM((ep,), jnp.int32),
            pltpu.VMEM((ep,), jnp.int32),
            pltpu.VMEM((ep,), jnp.float32),
            pltpu.VMEM((G,), jnp.int32),
            pltpu.VMEM((G,), jnp.int32),
            pltpu.VMEM((G,), jnp.int32),
            pltpu.VMEM((G,), jnp.int32),
            pltpu.VMEM((G,), jnp.int32),
            pltpu.VMEM((G,), jnp.int32),
            pltpu.VMEM((1, G), jnp.int32),
            pltpu.VMEM_SHARED((G,), jnp.int32),
        ],
    )
    def k(key_hbm, seg_hbm, kper_hbm, sc_hbm, samp_hbm, ewb_hbm, key_v,
          seg_v, sco_v, kp_v, lo_v, hi_v, mid_v, h_v, cnt_v, idn_v, cnt_sp):
        cid = lax.axis_index("c")
        sid = lax.axis_index("s")
        base = sid * ep
        pltpu.sync_copy(key_hbm.at[pl.ds(base, ep)], key_v)
        pltpu.sync_copy(seg_hbm.at[pl.ds(base, ep)], seg_v)
        pltpu.sync_copy(sc_hbm.at[pl.ds(base, ep)], sco_v)
        pltpu.sync_copy(kper_hbm, kp_v)
        for j in range(G // 16):
            sl = pl.ds(16 * j, 16)
            lo_v[sl] = jnp.full((16,), INT_MIN, jnp.int32)
            hi_v[sl] = jnp.full((16,), 2147483647, jnp.int32)
            idn_v[0, sl] = _i16() + 16 * j
            h_v[sl] = jnp.zeros((16,), jnp.int32)

        @pl.when(sid == 0)
        def _():
            pltpu.sync_copy(h_v, cnt_sp)

        plsc.subcore_barrier()
        ones16 = jnp.full((16,), 1, jnp.int32)

        def it(t, _):
            for j in range(G // 16):
                sl = pl.ds(16 * j, 16)
                lo16 = lo_v[sl]
                hi16 = hi_v[sl]
                mid_v[sl] = ((lo16 >> 1) + (hi16 >> 1) +
                             (lo16 & hi16 & jnp.int32(1)))
                h_v[sl] = jnp.zeros((16,), jnp.int32)

            def cnt1(g, _):
                idx = _i16() + 16 * g
                k16 = plsc.load_gather(key_v, [idx])
                s16 = plsc.load_gather(seg_v, [idx])
                m16 = plsc.load_gather(mid_v, [s16])
                plsc.addupdate_scatter(h_v, [s16], ones16, mask=k16 > m16)
                return 0

            lax.fori_loop(0, ngr, cnt1, 0)
            pltpu.sync_copy(h_v, cnt_sp.at[idn_v.at[0]], add=True)
            plsc.subcore_barrier()
            pltpu.sync_copy(cnt_sp, cnt_v)
            plsc.subcore_barrier()

            @pl.when(sid == 0)
            def _():
                for j in range(G // 16):
                    h_v[pl.ds(16 * j, 16)] = jnp.zeros((16,), jnp.int32)
                pltpu.sync_copy(h_v, cnt_sp)

            plsc.subcore_barrier()
            for j in range(G // 16):
                sl = pl.ds(16 * j, 16)
                ge = cnt_v[sl] >= kp_v[sl]
                m16 = mid_v[sl]
                lo_v[sl] = jnp.where(ge, m16, lo_v[sl])
                hi_v[sl] = jnp.where(ge, hi_v[sl], m16)
            return 0

        lax.fori_loop(0, 32, it, 0)

        @pl.when(cid == 0)
        def _():
            def fin(g, _):
                idx = _i16() + 16 * g
                k16 = plsc.load_gather(key_v, [idx])
                s16 = plsc.load_gather(seg_v, [idx])
                sc16 = plsc.load_gather(sco_v, [idx])
                lo16 = plsc.load_gather(lo_v, [s16])
                kp16 = plsc.load_gather(kp_v, [s16])
                sel = (k16 > lo16) & (kp16 > 0)
                samp = jnp.where(sel, 1.0, 0.0).astype(jnp.float32)
                plsc.store_scatter(sco_v, [idx], samp)
                plsc.store_scatter(key_v, [idx],
                                   plsc.bitcast(sc16 * samp, jnp.int32))
                return 0

            lax.fori_loop(0, ngr, fin, 0)
            pltpu.sync_copy(sco_v, samp_hbm.at[pl.ds(base, ep)])
            pltpu.sync_copy(key_v, ewb_hbm.at[pl.ds(base, ep)])

    return k(key, seg, kper, scores)


def _tc_mm(x, w):
    def body(x_ref, w_ref, o_ref):
        o_ref[...] = jnp.dot(x_ref[...], w_ref[...],
                             preferred_element_type=jnp.float32)

    return pl.pallas_call(
        body, out_shape=jax.ShapeDtypeStruct((x.shape[0], w.shape[1]),
                                             jnp.float32))(x, w)


def _tc_dinv(degf):
    rows = degf.shape[0] // 2

    def body(d_ref, o_ref):
        d = 1.0 + d_ref[:rows, :] + d_ref[rows:, :]
        o_ref[...] = jnp.where(d > 0, lax.rsqrt(jnp.where(d > 0, d, 1.0)),
                               0.0)

    return pl.pallas_call(
        body, out_shape=jax.ShapeDtypeStruct((rows, 128), jnp.float32))(degf)


def _tc_scale(hw, dinv_col):
    def body(h_ref, s_ref, o_ref):
        o_ref[...] = h_ref[...] * s_ref[...]

    return pl.pallas_call(
        body, out_shape=jax.ShapeDtypeStruct(hw.shape, jnp.float32))(
            hw, dinv_col)


def _tc_post(acc, hw, dinv_col, b_row, w_next, n_pad, want_h, want_g):
    n = hw.shape[0]

    def body(a_ref, h_ref, s_ref, b_ref, w_ref, *outs):
        dinv = s_ref[...]
        tot = a_ref[:n, :] + a_ref[n_pad:n_pad + n, :]
        h = jnp.maximum(dinv * tot + dinv * dinv * h_ref[...] + b_ref[...],
                        0.0)
        hw_n = jnp.dot(h, w_ref[...], preferred_element_type=jnp.float32)
        i = 0
        if want_h:
            outs[i][...] = h
            i += 1
        outs[i][...] = hw_n
        i += 1
        if want_g:
            outs[i][...] = dinv * hw_n

    shapes = []
    if want_h:
        shapes.append(jax.ShapeDtypeStruct((n, 128), jnp.float32))
    shapes.append(jax.ShapeDtypeStruct((n, 128), jnp.float32))
    if want_g:
        shapes.append(jax.ShapeDtypeStruct((n, 128), jnp.float32))
    return pl.pallas_call(body, out_shape=tuple(shapes))(
        acc, hw, dinv_col, b_row, w_next)


def _tc_keyprep(sc2, u2, src2, dst2, cnts4, ratio11, train11):
    def body(s_ref, u_ref, a_ref, b_ref, c_ref, r_ref, t_ref, key_ref,
             kp_ref, sec_ref):
        s = s_ref[...]
        gum = -jnp.log(-jnp.log(u_ref[...]))
        train = t_ref[0, 0] != 0
        pert = jnp.where(train, s + gum, s)
        ib = lax.bitcast_convert_type(pert, jnp.int32)
        ik = ib ^ ((ib >> 31) & jnp.int32(0x7FFFFFFF))
        key_ref[...] = jnp.where(a_ref[...] < b_ref[...], ik, INT_MIN)
        tot = c_ref[0:1, :] + c_ref[2:3, :]
        msk = c_ref[1:2, :] + c_ref[3:4, :]
        sec_ref[...] = tot
        kp_ref[...] = jnp.floor(r_ref[0, 0] *
                                msk.astype(jnp.float32)).astype(jnp.int32)

    return pl.pallas_call(
        body,
        out_shape=(jax.ShapeDtypeStruct(sc2.shape, jnp.int32),
                   jax.ShapeDtypeStruct((1, G), jnp.int32),
                   jax.ShapeDtypeStruct((1, G), jnp.int32)))(
            sc2, u2, src2, dst2, cnts4, ratio11, train11)


def _tc_final(acc, hw, dinv_col, b_row, batch_col, l1w, l1b, l2wp, l2bp,
              n_pad):
    n = hw.shape[0]

    def body(a_ref, h_ref, s_ref, b_ref, bc_ref, w1_ref, b1_ref, w2_ref,
             b2_ref, o_ref):
        dinv = s_ref[...]
        tot = a_ref[:n, :] + a_ref[n_pad:n_pad + n, :]
        h3 = jnp.maximum(dinv * tot + dinv * dinv * h_ref[...] + b_ref[...],
                         0.0)
        oh = (bc_ref[...] == lax.broadcasted_iota(jnp.int32, (n, 128),
                                                  1)).astype(jnp.float32)
        pooled_t = lax.dot_general(oh, h3, (((0,), (0,)), ((), ())),
                                   preferred_element_type=jnp.float32)
        cnt = jnp.sum(oh, axis=0, keepdims=True)
        recip = 1.0 / jnp.maximum(cnt, 1.0)
        ri = lax.broadcasted_iota(jnp.int32, (128, 128), 0)
        ci = lax.broadcasted_iota(jnp.int32, (128, 128), 1)
        diag = jnp.where(ri == ci, jnp.broadcast_to(recip, (128, 128)), 0.0)
        pooled = jnp.dot(diag, pooled_t, preferred_element_type=jnp.float32)
        z1 = jnp.maximum(
            jnp.dot(pooled, w1_ref[...], preferred_element_type=jnp.float32)
            + b1_ref[...], 0.0)
        z2 = jnp.dot(z1, w2_ref[...],
                     preferred_element_type=jnp.float32) + b2_ref[...]
        m = jnp.max(z2, axis=1, keepdims=True)
        ez = jnp.exp(z2 - m)
        o_ref[...] = z2 - m - jnp.log(jnp.sum(ez, axis=1, keepdims=True))

    return pl.pallas_call(
        body, out_shape=jax.ShapeDtypeStruct((128, 128), jnp.float32))(
            acc, hw, dinv_col, b_row, batch_col, l1w, l1b, l2wp, l2bp)


def kernel(x, edge_index, batch, ratio, train_phase, W0, b0, W1, b1, W2, b2,
           lin1_W, lin1_b, lin2_W, lin2_b):
    n, d = x.shape
    e = edge_index.shape[1]
    src = edge_index[0]
    dst = edge_index[1]
    n_pad = ((n + NS * CH - 1) // (NS * CH)) * (NS * CH)

    u = jax.random.uniform(jax.random.key(42), (e,), minval=1e-6,
                           maxval=1.0 - 1e-6)
    ones_e = jnp.ones((e,), jnp.float32)
    b0r = b0.reshape(1, d)
    b1r = b1.reshape(1, d)
    b2r = b2.reshape(1, d)
    l1br = lin1_b.reshape(1, d)
    c = lin2_W.shape[1]
    l2wp = jnp.pad(lin2_W, ((0, 0), (0, 128 - c)))
    l2bp = jnp.concatenate(
        [lin2_b, jnp.full((128 - c,), -1e30, jnp.float32)]).reshape(1, 128)
    ratio11 = jnp.asarray(ratio, jnp.float32).reshape(1, 1)
    train11 = jnp.asarray(train_phase, jnp.int32).reshape(1, 1)

    hw0 = _tc_mm(x, W0)
    degp1 = _sc_deg(dst, ones_e, n_pad)
    dinv1 = _tc_dinv(degp1.reshape(2 * n_pad // 128, 128))
    dinv1_col = dinv1.reshape(n_pad, 1)[:n]
    g1 = _tc_scale(hw0, dinv1_col)
    acc1 = _sc_rows(g1, src, dst, ones_e, n_pad, scaled=False)
    h1, hw1 = _tc_post(acc1, hw0, dinv1_col, b0r, W1, n_pad, want_h=True,
                       want_g=False)

    scores, seg, cnts = _sc_score(h1, src, dst, batch)
    key2, kp2, sec2 = _tc_keyprep(
        scores.reshape(e // 128, 128), u.reshape(e // 128, 128),
        src.reshape(e // 128, 128), dst.reshape(e // 128, 128),
        cnts.reshape(4, G), ratio11, train11)
    sampled, ewb = _sc_topk(key2.reshape(e), seg, kp2.reshape(G), scores)
    ew = lax.bitcast_convert_type(ewb, jnp.float32)

    degp2 = _sc_deg(dst, ew, n_pad)
    dinv2 = _tc_dinv(degp2.reshape(2 * n_pad // 128, 128))
    dinv2_col = dinv2.reshape(n_pad, 1)[:n]
    g2 = _tc_scale(hw1, dinv2_col)
    acc2 = _sc_rows(g2, src, dst, ew, n_pad, scaled=True)
    hw2, g3 = _tc_post(acc2, hw1, dinv2_col, b1r, W2, n_pad, want_h=False,
                       want_g=True)
    acc3 = _sc_rows(g3, src, dst, ew, n_pad, scaled=True)

    outp = _tc_final(acc3, hw2, dinv2_col, b2r, batch.reshape(n, 1), lin1_W,
                     l1br, l2wp, l2bp, n_pad)
    return outp[:G, :c], sampled, sec2.reshape(G)


# speedup vs baseline: 9.0010x; 1.3593x over previous
import functools

import jax
import jax.numpy as jnp
from jax import lax
from jax.experimental import pallas as pl
from jax.experimental.pallas import tpu as pltpu
from jax.experimental.pallas import tpu_sc as plsc

NC = 2
NS = 16
NW = NC * NS
G = 64
CH = 80
INT_MIN = -2147483648

_mesh = plsc.VectorSubcoreMesh(core_axis_name="c", subcore_axis_name="s",
                               num_cores=NC, num_subcores=NS)
_cparams = pltpu.CompilerParams(needs_layout_passes=False)

_i16 = lambda: lax.iota(jnp.int32, 16)


def _splat_i(v):
    return jnp.full((16,), 1, jnp.int32) * v


def _splat_f(v):
    return jnp.full((16,), 1.0, jnp.float32) * v


def _sc_deg(dst, ew, n_pad):
    e = dst.shape[0]
    ew_per = e // NW
    nch = ew_per // CH
    nps = n_pad // NS

    @functools.partial(
        pl.kernel, mesh=_mesh, compiler_params=_cparams,
        out_type=jax.ShapeDtypeStruct((NC * n_pad,), jnp.float32),
        scratch_types=[
            pltpu.VMEM((1, CH), jnp.int32),
            pltpu.VMEM((CH,), jnp.float32),
            pltpu.VMEM((nps,), jnp.float32),
            pltpu.VMEM_SHARED((n_pad,), jnp.float32),
        ],
    )
    def k(dst_hbm, ew_hbm, out_hbm, idx_v, val_v, z_v, deg_sp):
        cid = lax.axis_index("c")
        sid = lax.axis_index("s")
        wid = cid * NS + sid
        for j in range(nps // 16):
            z_v[pl.ds(16 * j, 16)] = jnp.zeros((16,), jnp.float32)
        pltpu.sync_copy(z_v, deg_sp.at[pl.ds(sid * nps, nps)])
        plsc.subcore_barrier()

        def body(c, _):
            off = wid * ew_per + c * CH
            pltpu.sync_copy(dst_hbm.at[pl.ds(off, CH)], idx_v.at[0])
            pltpu.sync_copy(ew_hbm.at[pl.ds(off, CH)], val_v)
            pltpu.sync_copy(val_v, deg_sp.at[idx_v.at[0]], add=True)
            return 0

        lax.fori_loop(0, nch, body, 0)
        plsc.subcore_barrier()
        pltpu.sync_copy(deg_sp.at[pl.ds(sid * nps, nps)],
                        out_hbm.at[pl.ds(cid * n_pad + sid * nps, nps)])

    return k(dst, ew)


def _sc_rows(g_nodes, src, dst, ew, n_pad, scaled):
    e = src.shape[0]
    d = g_nodes.shape[1]
    ew_per = e // NW
    nch = ew_per // CH
    nps = n_pad // NS
    nf = d // 16

    scratch = [
        pltpu.VMEM((CH,), jnp.int32),
        pltpu.VMEM((1, CH), jnp.int32),
        pltpu.VMEM((CH, d), jnp.float32),
        pltpu.VMEM((CH,), jnp.float32),
        pltpu.VMEM_SHARED((n_pad, d), jnp.float32),
        pltpu.SemaphoreType.DMA,
    ]
    if scaled:
        cap = ew_per + CH
        scratch += [
            pltpu.VMEM((cap,), jnp.int32),
            pltpu.VMEM((cap,), jnp.int32),
            pltpu.VMEM((cap,), jnp.float32),
        ]

    @functools.partial(
        pl.kernel, mesh=_mesh, compiler_params=_cparams,
        out_type=jax.ShapeDtypeStruct((NC * n_pad, d), jnp.float32),
        scratch_types=scratch,
    )
    def k(g_hbm, src_hbm, dst_hbm, ew_hbm, out_hbm, gidx_v, didx_v, rows_v,
          ew_v, acc_sp, sem, *comp):
        cid = lax.axis_index("c")
        sid = lax.axis_index("s")
        wid = cid * NS + sid

        def zrow(i, _):
            si = _splat_i(i)
            for f in range(nf):
                plsc.store_scatter(rows_v, [si, _i16() + 16 * f],
                                   jnp.zeros((16,), jnp.float32))
            return 0

        lax.fori_loop(0, CH, zrow, 0)
        for j in range(nps // CH):
            pltpu.sync_copy(rows_v, acc_sp.at[pl.ds(sid * nps + j * CH, CH)])
        plsc.subcore_barrier()

        if not scaled:
            def body(c, _):
                off = wid * ew_per + c * CH
                pltpu.sync_copy(src_hbm.at[pl.ds(off, CH)], gidx_v)
                pltpu.async_copy(g_hbm.at[gidx_v], rows_v, sem).wait()
                pltpu.sync_copy(dst_hbm.at[pl.ds(off, CH)], didx_v.at[0])
                pltpu.sync_copy(rows_v, acc_sp.at[didx_v.at[0]], add=True)
                return 0

            lax.fori_loop(0, nch, body, 0)
        else:
            csrc_v, cdst_v, cew_v = comp

            def compact(c, off):
                eoff = wid * ew_per + c * CH
                pltpu.sync_copy(src_hbm.at[pl.ds(eoff, CH)], gidx_v)
                pltpu.sync_copy(dst_hbm.at[pl.ds(eoff, CH)], didx_v.at[0])
                pltpu.sync_copy(ew_hbm.at[pl.ds(eoff, CH)], ew_v)
                for grp in range(CH // 16):
                    sl = pl.ds(16 * grp, 16)
                    wv = ew_v[sl]
                    m = wv != 0.0
                    didx = off + plsc.cumsum(m.astype(jnp.int32)) - 1
                    plsc.store_scatter(csrc_v, [didx], gidx_v[sl], mask=m)
                    plsc.store_scatter(cdst_v, [didx], didx_v[0, sl], mask=m)
                    plsc.store_scatter(cew_v, [didx], wv, mask=m)
                    off = off + jnp.max(
                        plsc.all_reduce_population_count(m))
                return off

            ncz = lax.fori_loop(0, nch, compact, jnp.int32(0))
            for grp in range(CH // 16):
                pidx = ncz + _i16() + 16 * grp
                plsc.store_scatter(csrc_v, [pidx], jnp.zeros((16,),
                                                            jnp.int32))
                plsc.store_scatter(cdst_v, [pidx],
                                   jnp.full((16,), n_pad - 1, jnp.int32))
                plsc.store_scatter(cew_v, [pidx], jnp.zeros((16,),
                                                            jnp.float32))

            def body(c, _):
                coff = c * CH
                pltpu.async_copy(g_hbm.at[csrc_v.at[pl.ds(coff, CH)]],
                                 rows_v, sem).wait()

                def scale(i, _):
                    si = _splat_i(i)
                    b = plsc.load_gather(cew_v, [_splat_i(coff) + si])
                    for f in range(nf):
                        col = _i16() + 16 * f
                        v = plsc.load_gather(rows_v, [si, col])
                        plsc.store_scatter(rows_v, [si, col], v * b)
                    return 0

                lax.fori_loop(0, CH, scale, 0, unroll=2)
                for grp in range(CH // 16):
                    didx_v[0, pl.ds(16 * grp, 16)] = plsc.load_gather(
                        cdst_v, [_splat_i(coff) + _i16() + 16 * grp])
                pltpu.sync_copy(rows_v, acc_sp.at[didx_v.at[0]], add=True)
                return 0

            lax.fori_loop(0, (ncz + CH - 1) // CH, body, 0)
        plsc.subcore_barrier()
        pltpu.sync_copy(acc_sp.at[pl.ds(sid * nps, nps)],
                        out_hbm.at[pl.ds(cid * n_pad + sid * nps, nps)])

    return k(g_nodes, src, dst, ew)


def _sc_score(h, src, dst, batch):
    e = src.shape[0]
    n, d = h.shape
    ew_per = e // NW
    nch = ew_per // CH
    nf = d // 16

    @functools.partial(
        pl.kernel, mesh=_mesh, compiler_params=_cparams,
        out_type=(jax.ShapeDtypeStruct((e,), jnp.float32),
                  jax.ShapeDtypeStruct((e,), jnp.int32),
                  jax.ShapeDtypeStruct((NC * 2 * G,), jnp.int32)),
        scratch_types=[
            pltpu.VMEM((CH,), jnp.int32),
            pltpu.VMEM((CH,), jnp.int32),
            pltpu.VMEM((CH, d), jnp.float32),
            pltpu.VMEM((CH, d), jnp.float32),
            pltpu.VMEM((CH,), jnp.float32),
            pltpu.VMEM((CH,), jnp.int32),
            pltpu.VMEM((n,), jnp.int32),
            pltpu.VMEM((2 * G,), jnp.int32),
            pltpu.VMEM((1, 2 * G), jnp.int32),
            pltpu.VMEM_SHARED((2 * G,), jnp.int32),
            pltpu.SemaphoreType.DMA,
        ],
    )
    def k(h_hbm, src_hbm, dst_hbm, batch_hbm, sc_hbm, seg_hbm, cnt_hbm,
          sidx_v, didx_v, ra_v, rb_v, sco_v, sgo_v, bt_v, hist_v, idn_v,
          cnt_sp, sem):
        cid = lax.axis_index("c")
        sid = lax.axis_index("s")
        wid = cid * NS + sid
        pltpu.sync_copy(batch_hbm, bt_v)
        for j in range(2 * G // 16):
            hist_v[pl.ds(16 * j, 16)] = jnp.zeros((16,), jnp.int32)
            idn_v[0, pl.ds(16 * j, 16)] = _i16() + 16 * j

        @pl.when(sid == 0)
        def _():
            pltpu.sync_copy(hist_v, cnt_sp)

        plsc.subcore_barrier()

        ones16 = jnp.full((16,), 1, jnp.int32)
        lane0 = _i16() == 0

        def body(c, _):
            off = wid * ew_per + c * CH
            pltpu.sync_copy(src_hbm.at[pl.ds(off, CH)], sidx_v)
            pltpu.sync_copy(dst_hbm.at[pl.ds(off, CH)], didx_v)
            pltpu.async_copy(h_hbm.at[sidx_v], ra_v, sem).wait()
            pltpu.async_copy(h_hbm.at[didx_v], rb_v, sem).wait()
            for grp in range(CH // 16):
                sv = sidx_v[pl.ds(16 * grp, 16)]
                dv = didx_v[pl.ds(16 * grp, 16)]
                seg16 = plsc.load_gather(bt_v, [sv])
                sgo_v[pl.ds(16 * grp, 16)] = seg16
                plsc.addupdate_scatter(hist_v, [seg16], ones16)
                plsc.addupdate_scatter(hist_v, [seg16 + G], ones16,
                                       mask=sv < dv)

            def dot1(i, _):
                si = _splat_i(i)
                acc = jnp.zeros((16,), jnp.float32)
                for f in range(nf):
                    col = _i16() + 16 * f
                    acc = acc + (plsc.load_gather(ra_v, [si, col]) *
                                 plsc.load_gather(rb_v, [si, col]))
                s = jnp.sum(acc)
                plsc.store_scatter(sco_v, [si], _splat_f(s), mask=lane0)
                return 0

            lax.fori_loop(0, CH, dot1, 0, unroll=4)
            pltpu.sync_copy(sco_v, sc_hbm.at[pl.ds(off, CH)])
            pltpu.sync_copy(sgo_v, seg_hbm.at[pl.ds(off, CH)])
            return 0

        lax.fori_loop(0, nch, body, 0)
        pltpu.sync_copy(hist_v, cnt_sp.at[idn_v.at[0]], add=True)
        plsc.subcore_barrier()

        @pl.when(sid == 0)
        def _():
            pltpu.sync_copy(cnt_sp, cnt_hbm.at[pl.ds(cid * 2 * G, 2 * G)])

    return k(h, src, dst, batch)


def _sc_topk(key, seg, kper, scores):
    e = key.shape[0]
    ep = e // NS
    ngr = ep // 16

    @functools.partial(
        pl.kernel, mesh=_mesh, compiler_params=_cparams,
        out_type=(jax.ShapeDtypeStruct((e,), jnp.float32),
                  jax.ShapeDtypeStruct((e,), jnp.int32)),
        scratch_types=[
            pltpu.VMEM((ep,), jnp.int32),
            pltpu.VMEM((ep,), jnp.int32),
            pltpu.VMEM((ep,), jnp.float32),
            pltpu.VMEM((G,), jnp.int32),
            pltpu.VMEM((G,), jnp.int32),
            pltpu.VMEM((G,), jnp.int32),
            pltpu.VMEM((G,), jnp.int32),
            pltpu.VMEM((G,), jnp.int32),
            pltpu.VMEM((G,), jnp.int32),
            pltpu.VMEM((1, G), jnp.int32),
            pltpu.VMEM_SHARED((G,), jnp.int32),
        ],
    )
    def k(key_hbm, seg_hbm, kper_hbm, sc_hbm, samp_hbm, ewb_hbm, key_v,
          seg_v, sco_v, kp_v, lo_v, hi_v, mid_v, h_v, cnt_v, idn_v, cnt_sp):
        cid = lax.axis_index("c")
        sid = lax.axis_index("s")
        base = sid * ep
        pltpu.sync_copy(key_hbm.at[pl.ds(base, ep)], key_v)
        pltpu.sync_copy(seg_hbm.at[pl.ds(base, ep)], seg_v)
        pltpu.sync_copy(sc_hbm.at[pl.ds(base, ep)], sco_v)
        pltpu.sync_copy(kper_hbm, kp_v)
        for j in range(G // 16):
            sl = pl.ds(16 * j, 16)
            lo_v[sl] = jnp.full((16,), INT_MIN, jnp.int32)
            hi_v[sl] = jnp.full((16,), 2147483647, jnp.int32)
            idn_v[0, sl] = _i16() + 16 * j
            h_v[sl] = jnp.zeros((16,), jnp.int32)

        @pl.when(sid == 0)
        def _():
            pltpu.sync_copy(h_v, cnt_sp)

        plsc.subcore_barrier()
        ones16 = jnp.full((16,), 1, jnp.int32)

        def it(t, _):
            for j in range(G // 16):
                sl = pl.ds(16 * j, 16)
                lo16 = lo_v[sl]
                hi16 = hi_v[sl]
                mid_v[sl] = ((lo16 >> 1) + (hi16 >> 1) +
                             (lo16 & hi16 & jnp.int32(1)))
                h_v[sl] = jnp.zeros((16,), jnp.int32)

            def cnt1(g, _):
                idx = _i16() + 16 * g
                k16 = plsc.load_gather(key_v, [idx])
                s16 = plsc.load_gather(seg_v, [idx])
                m16 = plsc.load_gather(mid_v, [s16])
                plsc.addupdate_scatter(h_v, [s16], ones16, mask=k16 > m16)
                return 0

            lax.fori_loop(0, ngr, cnt1, 0, unroll=4)
            pltpu.sync_copy(h_v, cnt_sp.at[idn_v.at[0]], add=True)
            plsc.subcore_barrier()
            pltpu.sync_copy(cnt_sp, cnt_v)
            plsc.subcore_barrier()

            @pl.when(sid == 0)
            def _():
                for j in range(G // 16):
                    h_v[pl.ds(16 * j, 16)] = jnp.zeros((16,), jnp.int32)
                pltpu.sync_copy(h_v, cnt_sp)

            plsc.subcore_barrier()
            for j in range(G // 16):
                sl = pl.ds(16 * j, 16)
                ge = cnt_v[sl] >= kp_v[sl]
                m16 = mid_v[sl]
                lo_v[sl] = jnp.where(ge, m16, lo_v[sl])
                hi_v[sl] = jnp.where(ge, hi_v[sl], m16)
            return 0

        lax.fori_loop(0, 32, it, 0)

        @pl.when(cid == 0)
        def _():
            def fin(g, _):
                idx = _i16() + 16 * g
                k16 = plsc.load_gather(key_v, [idx])
                s16 = plsc.load_gather(seg_v, [idx])
                sc16 = plsc.load_gather(sco_v, [idx])
                lo16 = plsc.load_gather(lo_v, [s16])
                kp16 = plsc.load_gather(kp_v, [s16])
                sel = (k16 > lo16) & (kp16 > 0)
                samp = jnp.where(sel, 1.0, 0.0).astype(jnp.float32)
                plsc.store_scatter(sco_v, [idx], samp)
                plsc.store_scatter(key_v, [idx],
                                   plsc.bitcast(sc16 * samp, jnp.int32))
                return 0

            lax.fori_loop(0, ngr, fin, 0)
            pltpu.sync_copy(sco_v, samp_hbm.at[pl.ds(base, ep)])
            pltpu.sync_copy(key_v, ewb_hbm.at[pl.ds(base, ep)])

    return k(key, seg, kper, scores)


def _tc_mm(x, w):
    def body(x_ref, w_ref, o_ref):
        o_ref[...] = jnp.dot(x_ref[...], w_ref[...],
                             preferred_element_type=jnp.float32)

    return pl.pallas_call(
        body, out_shape=jax.ShapeDtypeStruct((x.shape[0], w.shape[1]),
                                             jnp.float32))(x, w)


def _tc_dinv(degf):
    rows = degf.shape[0] // 2

    def body(d_ref, o_ref):
        d = 1.0 + d_ref[:rows, :] + d_ref[rows:, :]
        o_ref[...] = jnp.where(d > 0, lax.rsqrt(jnp.where(d > 0, d, 1.0)),
                               0.0)

    return pl.pallas_call(
        body, out_shape=jax.ShapeDtypeStruct((rows, 128), jnp.float32))(degf)


def _tc_scale(hw, dinv_col):
    def body(h_ref, s_ref, o_ref):
        o_ref[...] = h_ref[...] * s_ref[...]

    return pl.pallas_call(
        body, out_shape=jax.ShapeDtypeStruct(hw.shape, jnp.float32))(
            hw, dinv_col)


def _tc_post(acc, hw, dinv_col, b_row, w_next, n_pad, want_h, want_g):
    n = hw.shape[0]

    def body(a_ref, h_ref, s_ref, b_ref, w_ref, *outs):
        dinv = s_ref[...]
        tot = a_ref[:n, :] + a_ref[n_pad:n_pad + n, :]
        h = jnp.maximum(dinv * tot + dinv * dinv * h_ref[...] + b_ref[...],
                        0.0)
        hw_n = jnp.dot(h, w_ref[...], preferred_element_type=jnp.float32)
        i = 0
        if want_h:
            outs[i][...] = h
            i += 1
        outs[i][...] = hw_n
        i += 1
        if want_g:
            outs[i][...] = dinv * hw_n

    shapes = []
    if want_h:
        shapes.append(jax.ShapeDtypeStruct((n, 128), jnp.float32))
    shapes.append(jax.ShapeDtypeStruct((n, 128), jnp.float32))
    if want_g:
        shapes.append(jax.ShapeDtypeStruct((n, 128), jnp.float32))
    return pl.pallas_call(body, out_shape=tuple(shapes))(
        acc, hw, dinv_col, b_row, w_next)


def _tc_keyprep(sc2, u2, src2, dst2, cnts4, ratio11, train11):
    def body(s_ref, u_ref, a_ref, b_ref, c_ref, r_ref, t_ref, key_ref,
             kp_ref, sec_ref):
        s = s_ref[...]
        gum = -jnp.log(-jnp.log(u_ref[...]))
        train = t_ref[0, 0] != 0
        pert = jnp.where(train, s + gum, s)
        ib = lax.bitcast_convert_type(pert, jnp.int32)
        ik = ib ^ ((ib >> 31) & jnp.int32(0x7FFFFFFF))
        key_ref[...] = jnp.where(a_ref[...] < b_ref[...], ik, INT_MIN)
        tot = c_ref[0:1, :] + c_ref[2:3, :]
        msk = c_ref[1:2, :] + c_ref[3:4, :]
        sec_ref[...] = tot
        kp_ref[...] = jnp.floor(r_ref[0, 0] *
                                msk.astype(jnp.float32)).astype(jnp.int32)

    return pl.pallas_call(
        body,
        out_shape=(jax.ShapeDtypeStruct(sc2.shape, jnp.int32),
                   jax.ShapeDtypeStruct((1, G), jnp.int32),
                   jax.ShapeDtypeStruct((1, G), jnp.int32)))(
            sc2, u2, src2, dst2, cnts4, ratio11, train11)


def _tc_final(acc, hw, dinv_col, b_row, batch_col, l1w, l1b, l2wp, l2bp,
              n_pad):
    n = hw.shape[0]

    def body(a_ref, h_ref, s_ref, b_ref, bc_ref, w1_ref, b1_ref, w2_ref,
             b2_ref, o_ref):
        dinv = s_ref[...]
        tot = a_ref[:n, :] + a_ref[n_pad:n_pad + n, :]
        h3 = jnp.maximum(dinv * tot + dinv * dinv * h_ref[...] + b_ref[...],
                         0.0)
        oh = (bc_ref[...] == lax.broadcasted_iota(jnp.int32, (n, 128),
                                                  1)).astype(jnp.float32)
        pooled_t = lax.dot_general(oh, h3, (((0,), (0,)), ((), ())),
                                   preferred_element_type=jnp.float32)
        cnt = jnp.sum(oh, axis=0, keepdims=True)
        recip = 1.0 / jnp.maximum(cnt, 1.0)
        ri = lax.broadcasted_iota(jnp.int32, (128, 128), 0)
        ci = lax.broadcasted_iota(jnp.int32, (128, 128), 1)
        diag = jnp.where(ri == ci, jnp.broadcast_to(recip, (128, 128)), 0.0)
        pooled = jnp.dot(diag, pooled_t, preferred_element_type=jnp.float32)
        z1 = jnp.maximum(
            jnp.dot(pooled, w1_ref[...], preferred_element_type=jnp.float32)
            + b1_ref[...], 0.0)
        z2 = jnp.dot(z1, w2_ref[...],
                     preferred_element_type=jnp.float32) + b2_ref[...]
        m = jnp.max(z2, axis=1, keepdims=True)
        ez = jnp.exp(z2 - m)
        o_ref[...] = z2 - m - jnp.log(jnp.sum(ez, axis=1, keepdims=True))

    return pl.pallas_call(
        body, out_shape=jax.ShapeDtypeStruct((128, 128), jnp.float32))(
            acc, hw, dinv_col, b_row, batch_col, l1w, l1b, l2wp, l2bp)


def kernel(x, edge_index, batch, ratio, train_phase, W0, b0, W1, b1, W2, b2,
           lin1_W, lin1_b, lin2_W, lin2_b):
    n, d = x.shape
    e = edge_index.shape[1]
    src = edge_index[0]
    dst = edge_index[1]
    n_pad = ((n + NS * CH - 1) // (NS * CH)) * (NS * CH)

    u = jax.random.uniform(jax.random.key(42), (e,), minval=1e-6,
                           maxval=1.0 - 1e-6)
    ones_e = jnp.ones((e,), jnp.float32)
    b0r = b0.reshape(1, d)
    b1r = b1.reshape(1, d)
    b2r = b2.reshape(1, d)
    l1br = lin1_b.reshape(1, d)
    c = lin2_W.shape[1]
    l2wp = jnp.pad(lin2_W, ((0, 0), (0, 128 - c)))
    l2bp = jnp.concatenate(
        [lin2_b, jnp.full((128 - c,), -1e30, jnp.float32)]).reshape(1, 128)
    ratio11 = jnp.asarray(ratio, jnp.float32).reshape(1, 1)
    train11 = jnp.asarray(train_phase, jnp.int32).reshape(1, 1)

    hw0 = _tc_mm(x, W0)
    degp1 = _sc_deg(dst, ones_e, n_pad)
    dinv1 = _tc_dinv(degp1.reshape(2 * n_pad // 128, 128))
    dinv1_col = dinv1.reshape(n_pad, 1)[:n]
    g1 = _tc_scale(hw0, dinv1_col)
    acc1 = _sc_rows(g1, src, dst, ones_e, n_pad, scaled=False)
    h1, hw1 = _tc_post(acc1, hw0, dinv1_col, b0r, W1, n_pad, want_h=True,
                       want_g=False)

    scores, seg, cnts = _sc_score(h1, src, dst, batch)
    key2, kp2, sec2 = _tc_keyprep(
        scores.reshape(e // 128, 128), u.reshape(e // 128, 128),
        src.reshape(e // 128, 128), dst.reshape(e // 128, 128),
        cnts.reshape(4, G), ratio11, train11)
    sampled, ewb = _sc_topk(key2.reshape(e), seg, kp2.reshape(G), scores)
    ew = lax.bitcast_convert_type(ewb, jnp.float32)

    degp2 = _sc_deg(dst, ew, n_pad)
    dinv2 = _tc_dinv(degp2.reshape(2 * n_pad // 128, 128))
    dinv2_col = dinv2.reshape(n_pad, 1)[:n]
    g2 = _tc_scale(hw1, dinv2_col)
    acc2 = _sc_rows(g2, src, dst, ew, n_pad, scaled=True)
    hw2, g3 = _tc_post(acc2, hw1, dinv2_col, b1r, W2, n_pad, want_h=False,
                       want_g=True)
    acc3 = _sc_rows(g3, src, dst, ew, n_pad, scaled=True)

    outp = _tc_final(acc3, hw2, dinv2_col, b2r, batch.reshape(n, 1), lin1_W,
                     l1br, l2wp, l2bp, n_pad)
    return outp[:G, :c], sampled, sec2.reshape(G)


# speedup vs baseline: 9.3475x; 1.0385x over previous
import functools

import jax
import jax.numpy as jnp
from jax import lax
from jax.experimental import pallas as pl
from jax.experimental.pallas import tpu as pltpu
from jax.experimental.pallas import tpu_sc as plsc

NC = 2
NS = 16
NW = NC * NS
G = 64
CH = 80
INT_MIN = -2147483648

_mesh = plsc.VectorSubcoreMesh(core_axis_name="c", subcore_axis_name="s",
                               num_cores=NC, num_subcores=NS)
_cparams = pltpu.CompilerParams(needs_layout_passes=False)

_i16 = lambda: lax.iota(jnp.int32, 16)


def _splat_i(v):
    return jnp.full((16,), 1, jnp.int32) * v


def _splat_f(v):
    return jnp.full((16,), 1.0, jnp.float32) * v


def _sc_deg(dst, ew, n_pad):
    e = dst.shape[0]
    ew_per = e // NW
    nch = ew_per // CH
    nps = n_pad // NS

    @functools.partial(
        pl.kernel, mesh=_mesh, compiler_params=_cparams,
        out_type=jax.ShapeDtypeStruct((NC * n_pad,), jnp.float32),
        scratch_types=[
            pltpu.VMEM((1, CH), jnp.int32),
            pltpu.VMEM((CH,), jnp.float32),
            pltpu.VMEM((nps,), jnp.float32),
            pltpu.VMEM_SHARED((n_pad,), jnp.float32),
        ],
    )
    def k(dst_hbm, ew_hbm, out_hbm, idx_v, val_v, z_v, deg_sp):
        cid = lax.axis_index("c")
        sid = lax.axis_index("s")
        wid = cid * NS + sid
        for j in range(nps // 16):
            z_v[pl.ds(16 * j, 16)] = jnp.zeros((16,), jnp.float32)
        pltpu.sync_copy(z_v, deg_sp.at[pl.ds(sid * nps, nps)])
        plsc.subcore_barrier()

        def body(c, _):
            off = wid * ew_per + c * CH
            pltpu.sync_copy(dst_hbm.at[pl.ds(off, CH)], idx_v.at[0])
            pltpu.sync_copy(ew_hbm.at[pl.ds(off, CH)], val_v)
            pltpu.sync_copy(val_v, deg_sp.at[idx_v.at[0]], add=True)
            return 0

        lax.fori_loop(0, nch, body, 0)
        plsc.subcore_barrier()
        pltpu.sync_copy(deg_sp.at[pl.ds(sid * nps, nps)],
                        out_hbm.at[pl.ds(cid * n_pad + sid * nps, nps)])

    return k(dst, ew)


def _sc_rows(g_nodes, src, dst, ew, n_pad, scaled):
    e = src.shape[0]
    d = g_nodes.shape[1]
    ew_per = e // NW
    nch = ew_per // CH
    nps = n_pad // NS
    nf = d // 16

    scratch = [
        pltpu.VMEM((CH,), jnp.int32),
        pltpu.VMEM((1, CH), jnp.int32),
        pltpu.VMEM((CH, d), jnp.float32),
        pltpu.VMEM((CH,), jnp.float32),
        pltpu.VMEM_SHARED((n_pad, d), jnp.float32),
        pltpu.SemaphoreType.DMA,
    ]
    if scaled:
        cap = ew_per + CH
        scratch += [
            pltpu.VMEM((cap,), jnp.int32),
            pltpu.VMEM((cap,), jnp.int32),
            pltpu.VMEM((cap,), jnp.float32),
        ]

    @functools.partial(
        pl.kernel, mesh=_mesh, compiler_params=_cparams,
        out_type=jax.ShapeDtypeStruct((NC * n_pad, d), jnp.float32),
        scratch_types=scratch,
    )
    def k(g_hbm, src_hbm, dst_hbm, ew_hbm, out_hbm, gidx_v, didx_v, rows_v,
          ew_v, acc_sp, sem, *comp):
        cid = lax.axis_index("c")
        sid = lax.axis_index("s")
        wid = cid * NS + sid

        def zrow(i, _):
            si = _splat_i(i)
            for f in range(nf):
                plsc.store_scatter(rows_v, [si, _i16() + 16 * f],
                                   jnp.zeros((16,), jnp.float32))
            return 0

        lax.fori_loop(0, CH, zrow, 0)
        for j in range(nps // CH):
            pltpu.sync_copy(rows_v, acc_sp.at[pl.ds(sid * nps + j * CH, CH)])
        plsc.subcore_barrier()

        if not scaled:
            def body(c, _):
                off = wid * ew_per + c * CH
                pltpu.sync_copy(src_hbm.at[pl.ds(off, CH)], gidx_v)
                pltpu.async_copy(g_hbm.at[gidx_v], rows_v, sem).wait()
                pltpu.sync_copy(dst_hbm.at[pl.ds(off, CH)], didx_v.at[0])
                pltpu.sync_copy(rows_v, acc_sp.at[didx_v.at[0]], add=True)
                return 0

            lax.fori_loop(0, nch, body, 0)
        else:
            csrc_v, cdst_v, cew_v = comp

            def compact(c, off):
                eoff = wid * ew_per + c * CH
                pltpu.sync_copy(src_hbm.at[pl.ds(eoff, CH)], gidx_v)
                pltpu.sync_copy(dst_hbm.at[pl.ds(eoff, CH)], didx_v.at[0])
                pltpu.sync_copy(ew_hbm.at[pl.ds(eoff, CH)], ew_v)
                for grp in range(CH // 16):
                    sl = pl.ds(16 * grp, 16)
                    wv = ew_v[sl]
                    m = wv != 0.0
                    didx = off + plsc.cumsum(m.astype(jnp.int32)) - 1
                    plsc.store_scatter(csrc_v, [didx], gidx_v[sl], mask=m)
                    plsc.store_scatter(cdst_v, [didx], didx_v[0, sl], mask=m)
                    plsc.store_scatter(cew_v, [didx], wv, mask=m)
                    off = off + jnp.max(
                        plsc.all_reduce_population_count(m))
                return off

            ncz = lax.fori_loop(0, nch, compact, jnp.int32(0))
            for grp in range(CH // 16):
                pidx = ncz + _i16() + 16 * grp
                plsc.store_scatter(csrc_v, [pidx], jnp.zeros((16,),
                                                            jnp.int32))
                plsc.store_scatter(cdst_v, [pidx],
                                   jnp.full((16,), n_pad - 1, jnp.int32))
                plsc.store_scatter(cew_v, [pidx], jnp.zeros((16,),
                                                            jnp.float32))

            def body(c, _):
                coff = c * CH
                pltpu.async_copy(g_hbm.at[csrc_v.at[pl.ds(coff, CH)]],
                                 rows_v, sem).wait()

                def scale(i, _):
                    si = _splat_i(i)
                    b = plsc.load_gather(cew_v, [_splat_i(coff) + si])
                    for f in range(nf):
                        col = _i16() + 16 * f
                        v = plsc.load_gather(rows_v, [si, col])
                        plsc.store_scatter(rows_v, [si, col], v * b)
                    return 0

                lax.fori_loop(0, CH, scale, 0, unroll=2)
                for grp in range(CH // 16):
                    didx_v[0, pl.ds(16 * grp, 16)] = plsc.load_gather(
                        cdst_v, [_splat_i(coff) + _i16() + 16 * grp])
                pltpu.sync_copy(rows_v, acc_sp.at[didx_v.at[0]], add=True)
                return 0

            lax.fori_loop(0, (ncz + CH - 1) // CH, body, 0)
        plsc.subcore_barrier()
        pltpu.sync_copy(acc_sp.at[pl.ds(sid * nps, nps)],
                        out_hbm.at[pl.ds(cid * n_pad + sid * nps, nps)])

    return k(g_nodes, src, dst, ew)


def _sc_score(h, src, dst, batch):
    e = src.shape[0]
    n, d = h.shape
    ew_per = e // NW
    nch = ew_per // CH
    nf = d // 16

    @functools.partial(
        pl.kernel, mesh=_mesh, compiler_params=_cparams,
        out_type=(jax.ShapeDtypeStruct((e,), jnp.float32),
                  jax.ShapeDtypeStruct((e,), jnp.int32),
                  jax.ShapeDtypeStruct((NC * 2 * G,), jnp.int32)),
        scratch_types=[
            pltpu.VMEM((CH,), jnp.int32),
            pltpu.VMEM((CH,), jnp.int32),
            pltpu.VMEM((CH, d), jnp.float32),
            pltpu.VMEM((CH, d), jnp.float32),
            pltpu.VMEM((CH,), jnp.float32),
            pltpu.VMEM((CH,), jnp.int32),
            pltpu.VMEM((n,), jnp.int32),
            pltpu.VMEM((2 * G,), jnp.int32),
            pltpu.VMEM((1, 2 * G), jnp.int32),
            pltpu.VMEM_SHARED((2 * G,), jnp.int32),
            pltpu.SemaphoreType.DMA,
            pltpu.SemaphoreType.DMA,
        ],
    )
    def k(h_hbm, src_hbm, dst_hbm, batch_hbm, sc_hbm, seg_hbm, cnt_hbm,
          sidx_v, didx_v, ra_v, rb_v, sco_v, sgo_v, bt_v, hist_v, idn_v,
          cnt_sp, sem, sem2):
        cid = lax.axis_index("c")
        sid = lax.axis_index("s")
        wid = cid * NS + sid
        pltpu.sync_copy(batch_hbm, bt_v)
        for j in range(2 * G // 16):
            hist_v[pl.ds(16 * j, 16)] = jnp.zeros((16,), jnp.int32)
            idn_v[0, pl.ds(16 * j, 16)] = _i16() + 16 * j

        @pl.when(sid == 0)
        def _():
            pltpu.sync_copy(hist_v, cnt_sp)

        plsc.subcore_barrier()

        ones16 = jnp.full((16,), 1, jnp.int32)
        lane0 = _i16() == 0

        def body(c, _):
            off = wid * ew_per + c * CH
            pltpu.sync_copy(src_hbm.at[pl.ds(off, CH)], sidx_v)
            pltpu.sync_copy(dst_hbm.at[pl.ds(off, CH)], didx_v)
            cpa = pltpu.async_copy(h_hbm.at[sidx_v], ra_v, sem)
            cpb = pltpu.async_copy(h_hbm.at[didx_v], rb_v, sem2)
            cpa.wait()
            cpb.wait()
            for grp in range(CH // 16):
                sv = sidx_v[pl.ds(16 * grp, 16)]
                dv = didx_v[pl.ds(16 * grp, 16)]
                seg16 = plsc.load_gather(bt_v, [sv])
                sgo_v[pl.ds(16 * grp, 16)] = seg16
                plsc.addupdate_scatter(hist_v, [seg16], ones16)
                plsc.addupdate_scatter(hist_v, [seg16 + G], ones16,
                                       mask=sv < dv)

            def dot1(i, _):
                si = _splat_i(i)
                acc = jnp.zeros((16,), jnp.float32)
                for f in range(nf):
                    col = _i16() + 16 * f
                    acc = acc + (plsc.load_gather(ra_v, [si, col]) *
                                 plsc.load_gather(rb_v, [si, col]))
                s = jnp.sum(acc)
                plsc.store_scatter(sco_v, [si], _splat_f(s), mask=lane0)
                return 0

            lax.fori_loop(0, CH, dot1, 0, unroll=4)
            pltpu.sync_copy(sco_v, sc_hbm.at[pl.ds(off, CH)])
            pltpu.sync_copy(sgo_v, seg_hbm.at[pl.ds(off, CH)])
            return 0

        lax.fori_loop(0, nch, body, 0)
        pltpu.sync_copy(hist_v, cnt_sp.at[idn_v.at[0]], add=True)
        plsc.subcore_barrier()

        @pl.when(sid == 0)
        def _():
            pltpu.sync_copy(cnt_sp, cnt_hbm.at[pl.ds(cid * 2 * G, 2 * G)])

    return k(h, src, dst, batch)


def _sc_topk(key, seg, kper, scores):
    e = key.shape[0]
    ep = e // NS
    ngr = ep // 16

    @functools.partial(
        pl.kernel, mesh=_mesh, compiler_params=_cparams,
        out_type=(jax.ShapeDtypeStruct((e,), jnp.float32),
                  jax.ShapeDtypeStruct((e,), jnp.int32)),
        scratch_types=[
            pltpu.VMEM((ep,), jnp.int32),
            pltpu.VMEM((ep,), jnp.int32),
            pltpu.VMEM((ep,), jnp.float32),
            pltpu.VMEM((G,), jnp.int32),
            pltpu.VMEM((G,), jnp.int32),
            pltpu.VMEM((G,), jnp.int32),
            pltpu.VMEM((G,), jnp.int32),
            pltpu.VMEM((G,), jnp.int32),
            pltpu.VMEM((G,), jnp.int32),
            pltpu.VMEM((2, G), jnp.int32),
            pltpu.VMEM((G,), jnp.int32),
            pltpu.VMEM_SHARED((2 * G,), jnp.int32),
        ],
    )
    def k(key_hbm, seg_hbm, kper_hbm, sc_hbm, samp_hbm, ewb_hbm, key_v,
          seg_v, sco_v, kp_v, lo_v, hi_v, mid_v, h_v, cnt_v, idn_v, zb_v,
          cnt_sp):
        cid = lax.axis_index("c")
        sid = lax.axis_index("s")
        base = sid * ep
        pltpu.sync_copy(key_hbm.at[pl.ds(base, ep)], key_v)
        pltpu.sync_copy(seg_hbm.at[pl.ds(base, ep)], seg_v)
        pltpu.sync_copy(sc_hbm.at[pl.ds(base, ep)], sco_v)
        pltpu.sync_copy(kper_hbm, kp_v)
        for j in range(G // 16):
            sl = pl.ds(16 * j, 16)
            lo_v[sl] = jnp.full((16,), INT_MIN, jnp.int32)
            hi_v[sl] = jnp.full((16,), 2147483647, jnp.int32)
            idn_v[0, sl] = _i16() + 16 * j
            idn_v[1, sl] = _i16() + 16 * j + G
            zb_v[sl] = jnp.zeros((16,), jnp.int32)

        @pl.when(sid == 0)
        def _():
            pltpu.sync_copy(zb_v, cnt_sp.at[pl.ds(0, G)])
            pltpu.sync_copy(zb_v, cnt_sp.at[pl.ds(G, G)])

        plsc.subcore_barrier()
        ones16 = jnp.full((16,), 1, jnp.int32)

        def it(t, _):
            for j in range(G // 16):
                sl = pl.ds(16 * j, 16)
                lo16 = lo_v[sl]
                hi16 = hi_v[sl]
                mid_v[sl] = ((lo16 >> 1) + (hi16 >> 1) +
                             (lo16 & hi16 & jnp.int32(1)))
                h_v[sl] = jnp.zeros((16,), jnp.int32)

            def cnt1(g, _):
                idx = _i16() + 16 * g
                k16 = plsc.load_gather(key_v, [idx])
                s16 = plsc.load_gather(seg_v, [idx])
                m16 = plsc.load_gather(mid_v, [s16])
                plsc.addupdate_scatter(h_v, [s16], ones16, mask=k16 > m16)
                return 0

            lax.fori_loop(0, ngr, cnt1, 0, unroll=4)
            p = t & 1
            pltpu.sync_copy(h_v, cnt_sp.at[idn_v.at[p]], add=True)
            plsc.subcore_barrier()
            pltpu.sync_copy(cnt_sp.at[pl.ds(p * G, G)], cnt_v)

            @pl.when(sid == 0)
            def _():
                pltpu.sync_copy(zb_v, cnt_sp.at[pl.ds((1 - p) * G, G)])

            plsc.subcore_barrier()
            for j in range(G // 16):
                sl = pl.ds(16 * j, 16)
                ge = cnt_v[sl] >= kp_v[sl]
                m16 = mid_v[sl]
                lo_v[sl] = jnp.where(ge, m16, lo_v[sl])
                hi_v[sl] = jnp.where(ge, hi_v[sl], m16)
            return 0

        lax.fori_loop(0, 32, it, 0)

        @pl.when(cid == 0)
        def _():
            def fin(g, _):
                idx = _i16() + 16 * g
                k16 = plsc.load_gather(key_v, [idx])
                s16 = plsc.load_gather(seg_v, [idx])
                sc16 = plsc.load_gather(sco_v, [idx])
                lo16 = plsc.load_gather(lo_v, [s16])
                kp16 = plsc.load_gather(kp_v, [s16])
                sel = (k16 > lo16) & (kp16 > 0)
                samp = jnp.where(sel, 1.0, 0.0).astype(jnp.float32)
                plsc.store_scatter(sco_v, [idx], samp)
                plsc.store_scatter(key_v, [idx],
                                   plsc.bitcast(sc16 * samp, jnp.int32))
                return 0

            lax.fori_loop(0, ngr, fin, 0)
            pltpu.sync_copy(sco_v, samp_hbm.at[pl.ds(base, ep)])
            pltpu.sync_copy(key_v, ewb_hbm.at[pl.ds(base, ep)])

    return k(key, seg, kper, scores)


def _tc_mm(x, w):
    def body(x_ref, w_ref, o_ref):
        o_ref[...] = jnp.dot(x_ref[...], w_ref[...],
                             preferred_element_type=jnp.float32)

    return pl.pallas_call(
        body, out_shape=jax.ShapeDtypeStruct((x.shape[0], w.shape[1]),
                                             jnp.float32))(x, w)


def _tc_dinv(degf):
    rows = degf.shape[0] // 2

    def body(d_ref, o_ref):
        d = 1.0 + d_ref[:rows, :] + d_ref[rows:, :]
        o_ref[...] = jnp.where(d > 0, lax.rsqrt(jnp.where(d > 0, d, 1.0)),
                               0.0)

    return pl.pallas_call(
        body, out_shape=jax.ShapeDtypeStruct((rows, 128), jnp.float32))(degf)


def _tc_scale(hw, dinv_col):
    def body(h_ref, s_ref, o_ref):
        o_ref[...] = h_ref[...] * s_ref[...]

    return pl.pallas_call(
        body, out_shape=jax.ShapeDtypeStruct(hw.shape, jnp.float32))(
            hw, dinv_col)


def _tc_post(acc, hw, dinv_col, b_row, w_next, n_pad, want_h, want_g):
    n = hw.shape[0]

    def body(a_ref, h_ref, s_ref, b_ref, w_ref, *outs):
        dinv = s_ref[...]
        tot = a_ref[:n, :] + a_ref[n_pad:n_pad + n, :]
        h = jnp.maximum(dinv * tot + dinv * dinv * h_ref[...] + b_ref[...],
                        0.0)
        hw_n = jnp.dot(h, w_ref[...], preferred_element_type=jnp.float32)
        i = 0
        if want_h:
            outs[i][...] = h
            i += 1
        outs[i][...] = hw_n
        i += 1
        if want_g:
            outs[i][...] = dinv * hw_n

    shapes = []
    if want_h:
        shapes.append(jax.ShapeDtypeStruct((n, 128), jnp.float32))
    shapes.append(jax.ShapeDtypeStruct((n, 128), jnp.float32))
    if want_g:
        shapes.append(jax.ShapeDtypeStruct((n, 128), jnp.float32))
    return pl.pallas_call(body, out_shape=tuple(shapes))(
        acc, hw, dinv_col, b_row, w_next)


def _tc_keyprep(sc2, u2, src2, dst2, cnts4, ratio11, train11):
    def body(s_ref, u_ref, a_ref, b_ref, c_ref, r_ref, t_ref, key_ref,
             kp_ref, sec_ref):
        s = s_ref[...]
        gum = -jnp.log(-jnp.log(u_ref[...]))
        train = t_ref[0, 0] != 0
        pert = jnp.where(train, s + gum, s)
        ib = lax.bitcast_convert_type(pert, jnp.int32)
        ik = ib ^ ((ib >> 31) & jnp.int32(0x7FFFFFFF))
        key_ref[...] = jnp.where(a_ref[...] < b_ref[...], ik, INT_MIN)
        tot = c_ref[0:1, :] + c_ref[2:3, :]
        msk = c_ref[1:2, :] + c_ref[3:4, :]
        sec_ref[...] = tot
        kp_ref[...] = jnp.floor(r_ref[0, 0] *
                                msk.astype(jnp.float32)).astype(jnp.int32)

    return pl.pallas_call(
        body,
        out_shape=(jax.ShapeDtypeStruct(sc2.shape, jnp.int32),
                   jax.ShapeDtypeStruct((1, G), jnp.int32),
                   jax.ShapeDtypeStruct((1, G), jnp.int32)))(
            sc2, u2, src2, dst2, cnts4, ratio11, train11)


def _tc_final(acc, hw, dinv_col, b_row, batch_col, l1w, l1b, l2wp, l2bp,
              n_pad):
    n = hw.shape[0]

    def body(a_ref, h_ref, s_ref, b_ref, bc_ref, w1_ref, b1_ref, w2_ref,
             b2_ref, o_ref):
        dinv = s_ref[...]
        tot = a_ref[:n, :] + a_ref[n_pad:n_pad + n, :]
        h3 = jnp.maximum(dinv * tot + dinv * dinv * h_ref[...] + b_ref[...],
                         0.0)
        oh = (bc_ref[...] == lax.broadcasted_iota(jnp.int32, (n, 128),
                                                  1)).astype(jnp.float32)
        pooled_t = lax.dot_general(oh, h3, (((0,), (0,)), ((), ())),
                                   preferred_element_type=jnp.float32)
        cnt = jnp.sum(oh, axis=0, keepdims=True)
        recip = 1.0 / jnp.maximum(cnt, 1.0)
        ri = lax.broadcasted_iota(jnp.int32, (128, 128), 0)
        ci = lax.broadcasted_iota(jnp.int32, (128, 128), 1)
        diag = jnp.where(ri == ci, jnp.broadcast_to(recip, (128, 128)), 0.0)
        pooled = jnp.dot(diag, pooled_t, preferred_element_type=jnp.float32)
        z1 = jnp.maximum(
            jnp.dot(pooled, w1_ref[...], preferred_element_type=jnp.float32)
            + b1_ref[...], 0.0)
        z2 = jnp.dot(z1, w2_ref[...],
                     preferred_element_type=jnp.float32) + b2_ref[...]
        m = jnp.max(z2, axis=1, keepdims=True)
        ez = jnp.exp(z2 - m)
        o_ref[...] = z2 - m - jnp.log(jnp.sum(ez, axis=1, keepdims=True))

    return pl.pallas_call(
        body, out_shape=jax.ShapeDtypeStruct((128, 128), jnp.float32))(
            acc, hw, dinv_col, b_row, batch_col, l1w, l1b, l2wp, l2bp)


def kernel(x, edge_index, batch, ratio, train_phase, W0, b0, W1, b1, W2, b2,
           lin1_W, lin1_b, lin2_W, lin2_b):
    n, d = x.shape
    e = edge_index.shape[1]
    src = edge_index[0]
    dst = edge_index[1]
    n_pad = ((n + NS * CH - 1) // (NS * CH)) * (NS * CH)

    u = jax.random.uniform(jax.random.key(42), (e,), minval=1e-6,
                           maxval=1.0 - 1e-6)
    ones_e = jnp.ones((e,), jnp.float32)
    b0r = b0.reshape(1, d)
    b1r = b1.reshape(1, d)
    b2r = b2.reshape(1, d)
    l1br = lin1_b.reshape(1, d)
    c = lin2_W.shape[1]
    l2wp = jnp.pad(lin2_W, ((0, 0), (0, 128 - c)))
    l2bp = jnp.concatenate(
        [lin2_b, jnp.full((128 - c,), -1e30, jnp.float32)]).reshape(1, 128)
    ratio11 = jnp.asarray(ratio, jnp.float32).reshape(1, 1)
    train11 = jnp.asarray(train_phase, jnp.int32).reshape(1, 1)

    hw0 = _tc_mm(x, W0)
    degp1 = _sc_deg(dst, ones_e, n_pad)
    dinv1 = _tc_dinv(degp1.reshape(2 * n_pad // 128, 128))
    dinv1_col = dinv1.reshape(n_pad, 1)[:n]
    g1 = _tc_scale(hw0, dinv1_col)
    acc1 = _sc_rows(g1, src, dst, ones_e, n_pad, scaled=False)
    h1, hw1 = _tc_post(acc1, hw0, dinv1_col, b0r, W1, n_pad, want_h=True,
                       want_g=False)

    scores, seg, cnts = _sc_score(h1, src, dst, batch)
    key2, kp2, sec2 = _tc_keyprep(
        scores.reshape(e // 128, 128), u.reshape(e // 128, 128),
        src.reshape(e // 128, 128), dst.reshape(e // 128, 128),
        cnts.reshape(4, G), ratio11, train11)
    sampled, ewb = _sc_topk(key2.reshape(e), seg, kp2.reshape(G), scores)
    ew = lax.bitcast_convert_type(ewb, jnp.float32)

    degp2 = _sc_deg(dst, ew, n_pad)
    dinv2 = _tc_dinv(degp2.reshape(2 * n_pad // 128, 128))
    dinv2_col = dinv2.reshape(n_pad, 1)[:n]
    g2 = _tc_scale(hw1, dinv2_col)
    acc2 = _sc_rows(g2, src, dst, ew, n_pad, scaled=True)
    hw2, g3 = _tc_post(acc2, hw1, dinv2_col, b1r, W2, n_pad, want_h=False,
                       want_g=True)
    acc3 = _sc_rows(g3, src, dst, ew, n_pad, scaled=True)

    outp = _tc_final(acc3, hw2, dinv2_col, b2r, batch.reshape(n, 1), lin1_W,
                     l1br, l2wp, l2bp, n_pad)
    return outp[:G, :c], sampled, sec2.reshape(G)
